# baseline (speedup 1.0000x reference)
import functools
import itertools

import jax
import jax.numpy as jnp
from jax import lax
from jax.experimental import pallas as pl
from jax.experimental.pallas import tpu as pltpu
from jax.experimental.pallas import tpu_sc as plsc

N_ROWS = 16384
N_COLS = 4096
NC = 2
NS = 16
NW = NC * NS
R = N_ROWS // NW
L = 16
G = R // L
NCORNER = 16
CHUNK = 128

OFFS = tuple(d0 * 512 + d1 * 64 + d2 * 8 + d3
             for d0, d1, d2, d3 in itertools.product((0, 1), repeat=4))


def _interp_body(coords_hbm, mesh_hbm, out_hbm,
                 coords_v, cf_v, idx_v, vals_v, out_v, sem):
    cid = lax.axis_index("c")
    sid = lax.axis_index("s")
    wid = sid * NC + cid
    base_row = wid * R

    pltpu.sync_copy(coords_hbm.at[:, pl.ds(base_row, R)], coords_v)

    lane = lax.iota(jnp.int32, L)
    lane16 = lane * NCORNER

    copies = []
    for g in range(G):
        o = g * L
        fb = (base_row + o + lane) * N_COLS
        for d in range(4):
            c = coords_v[d, pl.ds(o, L)] * 7.0
            ci = c.astype(jnp.int32)
            ci = jnp.maximum(ci, 0)
            ci = jnp.minimum(ci, 6)
            cf_v[d, pl.ds(o, L)] = c - ci.astype(jnp.float32)
            fb = fb + ci * (512, 64, 8, 1)[d]
        for corner in range(NCORNER):
            plsc.store_scatter(idx_v, [lane16 + (o * NCORNER + corner)],
                               fb + OFFS[corner])
        for h in range(2):
            j = 2 * g + h
            copies.append(
                pltpu.async_copy(
                    mesh_hbm.at[idx_v.at[pl.ds(j * CHUNK, CHUNK)]],
                    vals_v.at[pl.ds(j * CHUNK, CHUNK)],
                    sem,
                )
            )
    for cp in copies:
        cp.wait()

    for g in range(G):
        o = g * L
        cf0 = cf_v[0, pl.ds(o, L)]
        cf1 = cf_v[1, pl.ds(o, L)]
        cf2 = cf_v[2, pl.ds(o, L)]
        cf3 = cf_v[3, pl.ds(o, L)]
        w01 = [[(1.0 - cf0) * (1.0 - cf1), (1.0 - cf0) * cf1],
               [cf0 * (1.0 - cf1), cf0 * cf1]]
        w23 = [[(1.0 - cf2) * (1.0 - cf3), (1.0 - cf2) * cf3],
               [cf2 * (1.0 - cf3), cf2 * cf3]]
        acc = None
        for corner, (d0, d1, d2, d3) in enumerate(
                itertools.product((0, 1), repeat=4)):
            v = plsc.load_gather(vals_v, [lane16 + (o * NCORNER + corner)])
            term = v * (w01[d0][d1] * w23[d2][d3])
            acc = term if acc is None else acc + term
        out_v[pl.ds(o, L)] = acc

    pltpu.sync_copy(out_v, out_hbm.at[pl.ds(base_row, R)])


_interp_kernel = functools.partial(
    pl.kernel,
    out_type=jax.ShapeDtypeStruct((N_ROWS,), jnp.float32),
    mesh=plsc.VectorSubcoreMesh(core_axis_name="c", subcore_axis_name="s"),
    compiler_params=pltpu.CompilerParams(needs_layout_passes=False),
    scratch_types=[
        pltpu.VMEM((4, R), jnp.float32),
        pltpu.VMEM((4, R), jnp.float32),
        pltpu.VMEM((R * NCORNER,), jnp.int32),
        pltpu.VMEM((R * NCORNER,), jnp.float32),
        pltpu.VMEM((R,), jnp.float32),
        pltpu.SemaphoreType.DMA,
    ],
)(_interp_body)


def kernel(coordinates, mesh_pred):
    coords_t = coordinates.T.reshape(4, N_ROWS)
    mesh_flat = mesh_pred.reshape(N_ROWS * N_COLS)
    return _interp_kernel(coords_t, mesh_flat)


# speedup vs baseline: 1.9365x; 1.9365x over previous
import functools
import itertools

import jax
import jax.numpy as jnp
from jax import lax
from jax.experimental import pallas as pl
from jax.experimental.pallas import tpu as pltpu
from jax.experimental.pallas import tpu_sc as plsc

N_ROWS = 16384
N_COLS = 4096
NC = 2
NW = 32
R = N_ROWS // NW
L = 16
G = R // L
NSLAB = R // 8
NSLOT = 3

OFFS = tuple(d1 * 64 + d2 * 8 + d3
             for d1, d2, d3 in itertools.product((0, 1), repeat=3))


def _interp_body(coords_hbm, mesh_hbm, out_hbm,
                 coords_v, cf0_v, cf1_v, cf2_v, cf3_v, cb_v,
                 slab_v, tmp_v, out_v, sem):
    cid = lax.axis_index("c")
    sid = lax.axis_index("s")
    wid = sid * NC + cid
    base_row = wid * R

    pltpu.sync_copy(coords_hbm.at[:, pl.ds(base_row, R)], coords_v)

    lane = lax.iota(jnp.int32, L)

    cf_refs = (cf0_v, cf1_v, cf2_v, cf3_v)
    for g in range(G):
        o = g * L
        cb = jnp.zeros((L,), jnp.int32)
        for d in range(4):
            c = coords_v[d, pl.ds(o, L)] * 7.0
            ci = c.astype(jnp.int32)
            ci = jnp.maximum(ci, 0)
            ci = jnp.minimum(ci, 6)
            cf_refs[d][pl.ds(o, L)] = c - ci.astype(jnp.float32)
            cb = cb + ci * (512, 64, 8, 1)[d]
        cb_v[pl.ds(o, L)] = cb

    def fire(k):
        slot = k % NSLOT
        return pltpu.async_copy(
            mesh_hbm.at[pl.ds(base_row + k * 8, 8), :],
            slab_v.at[pl.ds(slot * 8, 8)],
            sem,
        )

    copies = [None] * NSLAB
    for k in range(NSLOT):
        copies[k] = fire(k)

    rl = lane & 7
    d0sel = lane >= 8
    pend = None
    for k in range(NSLAB):
        copies[k].wait()
        slot = k % NSLOT
        rowg = k * 8 + rl
        cf0 = plsc.load_gather(cf0_v, [rowg])
        cf1 = plsc.load_gather(cf1_v, [rowg])
        cf2 = plsc.load_gather(cf2_v, [rowg])
        cf3 = plsc.load_gather(cf3_v, [rowg])
        cb = plsc.load_gather(cb_v, [rowg])
        f0 = jnp.where(d0sel, cf0, 1.0 - cf0)
        bc = cb + jnp.where(d0sel, 512, 0)
        g1, g2, g3 = 1.0 - cf1, 1.0 - cf2, 1.0 - cf3
        fw = [[f0 * g1 * g2, f0 * g1 * cf2], [f0 * cf1 * g2, f0 * cf1 * cf2]]
        acc = None
        srow = slot * 8 + rl
        for j, (d1, d2, d3) in enumerate(itertools.product((0, 1), repeat=3)):
            v = plsc.load_gather(slab_v, [srow, bc + OFFS[j]])
            term = v * (fw[d1][d2] * (cf3 if d3 else g3))
            acc = term if acc is None else acc + term
        if k + NSLOT < NSLAB:
            copies[k + NSLOT] = fire(k + NSLOT)
        tmp_v[...] = acc
        rowsum = (plsc.load_gather(tmp_v, [rl])
                  + plsc.load_gather(tmp_v, [rl + 8]))
        if pend is None:
            pend = rowsum
        else:
            out_v[pl.ds((k - 1) * 8, L)] = jnp.where(lane < 8, pend, rowsum)
            pend = None

    pltpu.sync_copy(out_v, out_hbm.at[pl.ds(base_row, R)])


_interp_kernel = functools.partial(
    pl.kernel,
    out_type=jax.ShapeDtypeStruct((N_ROWS,), jnp.float32),
    mesh=plsc.VectorSubcoreMesh(core_axis_name="c", subcore_axis_name="s"),
    compiler_params=pltpu.CompilerParams(needs_layout_passes=False),
    scratch_types=[
        pltpu.VMEM((4, R), jnp.float32),
        pltpu.VMEM((R,), jnp.float32),
        pltpu.VMEM((R,), jnp.float32),
        pltpu.VMEM((R,), jnp.float32),
        pltpu.VMEM((R,), jnp.float32),
        pltpu.VMEM((R,), jnp.int32),
        pltpu.VMEM((NSLOT * 8, N_COLS), jnp.float32),
        pltpu.VMEM((L,), jnp.float32),
        pltpu.VMEM((R,), jnp.float32),
        pltpu.SemaphoreType.DMA,
    ],
)(_interp_body)


def kernel(coordinates, mesh_pred):
    coords_t = coordinates.T.reshape(4, N_ROWS)
    return _interp_kernel(coords_t, mesh_pred)


# speedup vs baseline: 2.0370x; 1.0519x over previous
import functools
import itertools

import jax
import jax.numpy as jnp
from jax import lax
from jax.experimental import pallas as pl
from jax.experimental.pallas import tpu as pltpu
from jax.experimental.pallas import tpu_sc as plsc

N_ROWS = 16384
N_COLS = 4096
NC = 2
NW = 32
R = N_ROWS // NW
L = 16
G = R // L
NSLAB = R // 8
NSLOT = 3

OFFS = tuple(d1 * 64 + d2 * 8 + d3
             for d1, d2, d3 in itertools.product((0, 1), repeat=3))


def _interp_body(coords_hbm, mesh_hbm, out_hbm,
                 coords_v, cf0_v, cf1_v, cf2_v, cf3_v, cb_v,
                 slab_v, tmp_v, out_v, sem):
    cid = lax.axis_index("c")
    sid = lax.axis_index("s")
    wid = sid * NC + cid
    base_row = wid * R

    def fire(k):
        return pltpu.async_copy(
            mesh_hbm.at[pl.ds(base_row + k * 8, 8), :],
            slab_v.at[pl.ds((k % NSLOT) * 8, 8)],
            sem,
        )

    copies = [None] * NSLAB
    copies[0] = fire(0)
    copies[1] = fire(1)

    pltpu.sync_copy(coords_hbm.at[:, pl.ds(base_row, R)], coords_v)

    lane = lax.iota(jnp.int32, L)

    cf_refs = (cf0_v, cf1_v, cf2_v, cf3_v)
    for g in range(G):
        o = g * L
        cb = jnp.zeros((L,), jnp.int32)
        for d in range(4):
            c = coords_v[d, pl.ds(o, L)] * 7.0
            ci = c.astype(jnp.int32)
            ci = jnp.maximum(ci, 0)
            ci = jnp.minimum(ci, 6)
            cf_refs[d][pl.ds(o, L)] = c - ci.astype(jnp.float32)
            cb = cb + ci * (512, 64, 8, 1)[d]
        cb_v[pl.ds(o, L)] = cb

    rl = lane & 7
    d0sel = lane >= 8
    pend = None
    for k in range(NSLAB):
        copies[k].wait()
        if k + 2 < NSLAB:
            copies[k + 2] = fire(k + 2)
        slot = k % NSLOT
        rowg = k * 8 + rl
        cf0 = plsc.load_gather(cf0_v, [rowg])
        cf1 = plsc.load_gather(cf1_v, [rowg])
        cf2 = plsc.load_gather(cf2_v, [rowg])
        cf3 = plsc.load_gather(cf3_v, [rowg])
        cb = plsc.load_gather(cb_v, [rowg])
        f0 = jnp.where(d0sel, cf0, 1.0 - cf0)
        bc = cb + jnp.where(d0sel, 512, 0)
        g1, g2, g3 = 1.0 - cf1, 1.0 - cf2, 1.0 - cf3
        fw = [[f0 * g1 * g2, f0 * g1 * cf2], [f0 * cf1 * g2, f0 * cf1 * cf2]]
        acc = None
        srow = slot * 8 + rl
        for j, (d1, d2, d3) in enumerate(itertools.product((0, 1), repeat=3)):
            v = plsc.load_gather(slab_v, [srow, bc + OFFS[j]])
            term = v * (fw[d1][d2] * (cf3 if d3 else g3))
            acc = term if acc is None else acc + term
        tmp_v[...] = acc
        rowsum = (plsc.load_gather(tmp_v, [rl])
                  + plsc.load_gather(tmp_v, [rl + 8]))
        if pend is None:
            pend = rowsum
        else:
            out_v[pl.ds((k - 1) * 8, L)] = jnp.where(lane < 8, pend, rowsum)
            pend = None

    pltpu.sync_copy(out_v, out_hbm.at[pl.ds(base_row, R)])


_interp_kernel = functools.partial(
    pl.kernel,
    out_type=jax.ShapeDtypeStruct((N_ROWS,), jnp.float32),
    mesh=plsc.VectorSubcoreMesh(core_axis_name="c", subcore_axis_name="s"),
    compiler_params=pltpu.CompilerParams(needs_layout_passes=False),
    scratch_types=[
        pltpu.VMEM((4, R), jnp.float32),
        pltpu.VMEM((R,), jnp.float32),
        pltpu.VMEM((R,), jnp.float32),
        pltpu.VMEM((R,), jnp.float32),
        pltpu.VMEM((R,), jnp.float32),
        pltpu.VMEM((R,), jnp.int32),
        pltpu.VMEM((NSLOT * 8, N_COLS), jnp.float32),
        pltpu.VMEM((L,), jnp.float32),
        pltpu.VMEM((R,), jnp.float32),
        pltpu.SemaphoreType.DMA,
    ],
)(_interp_body)


def kernel(coordinates, mesh_pred):
    coords_t = coordinates.T.reshape(4, N_ROWS)
    return _interp_kernel(coords_t, mesh_pred)
